# baseline (speedup 1.0000x reference)
import functools

import jax
import jax.numpy as jnp
from jax import lax
from jax.experimental import pallas as pl
from jax.experimental.pallas import tpu as pltpu
from jax.experimental.pallas import tpu_sc as plsc

N = 10000
D = 128
E = 320000

NUM_CORES = 2
NUM_SUBCORES = 16
NW = NUM_CORES * NUM_SUBCORES

CHUNK = 128
K = 80
KB = 16
NBLK = K // KB
E_PAD = NW * K * CHUNK
NPAD = 10240
ROWS_PER_SUB = NPAD // NUM_SUBCORES
OUT_ROWS_PER_SUB = 624
OUT_TAIL = N - NUM_SUBCORES * OUT_ROWS_PER_SUB


def _sc_aggregate(feats, src2d, dst2d):
    mesh = plsc.VectorSubcoreMesh(core_axis_name="core",
                                  subcore_axis_name="subcore")

    @functools.partial(
        pl.kernel,
        out_type=jax.ShapeDtypeStruct((NUM_CORES, N, D), jnp.float32),
        mesh=mesh,
        scratch_types=[
            pltpu.VMEM((KB, CHUNK), jnp.int32),
            pltpu.VMEM((KB, CHUNK), jnp.int32),
            pltpu.VMEM((CHUNK, D), jnp.float32),
            pltpu.VMEM((CHUNK, D), jnp.float32),
            pltpu.VMEM_SHARED((NPAD, D), jnp.float32),
            pltpu.SemaphoreType.DMA,
            pltpu.SemaphoreType.DMA,
        ],
    )
    def k(feats_hbm, src_hbm, dst_hbm, out_hbm,
          src_v, dst_v, rows0_v, rows1_v, agg_sh, gsem0, gsem1):
        c = lax.axis_index("core")
        s = lax.axis_index("subcore")
        wid = c * NUM_SUBCORES + s

        @pl.loop(0, CHUNK)
        def _(i):
            @pl.loop(0, D, step=16)
            def _(j):
                rows0_v[i, pl.ds(j, 16)] = jnp.zeros((16,), jnp.float32)

        @pl.loop(0, ROWS_PER_SUB // CHUNK)
        def _(t):
            pltpu.sync_copy(rows0_v,
                            agg_sh.at[pl.ds(s * ROWS_PER_SUB + t * CHUNK,
                                            CHUNK)])

        plsc.subcore_barrier()

        @pl.loop(0, NBLK)
        def _(blk):
            base = wid * K + blk * KB
            pltpu.sync_copy(src_hbm.at[pl.ds(base, KB)], src_v)
            pltpu.sync_copy(dst_hbm.at[pl.ds(base, KB)], dst_v)

            pltpu.async_copy(feats_hbm.at[src_v.at[0]], rows0_v, gsem0)

            @pl.loop(0, KB, step=2)
            def _(j):
                pltpu.make_async_copy(feats_hbm.at[src_v.at[j]],
                                      rows0_v, gsem0).wait()
                pltpu.async_copy(feats_hbm.at[src_v.at[j + 1]],
                                 rows1_v, gsem1)
                pltpu.sync_copy(rows0_v, agg_sh.at[dst_v.at[j]], add=True)

                pltpu.make_async_copy(feats_hbm.at[src_v.at[j + 1]],
                                      rows1_v, gsem1).wait()

                @pl.when(j < KB - 2)
                def _():
                    pltpu.async_copy(feats_hbm.at[src_v.at[j + 2]],
                                     rows0_v, gsem0)

                pltpu.sync_copy(rows1_v, agg_sh.at[dst_v.at[j + 1]],
                                add=True)

        plsc.subcore_barrier()

        pltpu.sync_copy(agg_sh.at[pl.ds(s * OUT_ROWS_PER_SUB,
                                        OUT_ROWS_PER_SUB)],
                        out_hbm.at[c, pl.ds(s * OUT_ROWS_PER_SUB,
                                            OUT_ROWS_PER_SUB)])

        @pl.when(s == 0)
        def _():
            pltpu.sync_copy(
                agg_sh.at[pl.ds(NUM_SUBCORES * OUT_ROWS_PER_SUB, OUT_TAIL)],
                out_hbm.at[c, pl.ds(NUM_SUBCORES * OUT_ROWS_PER_SUB,
                                    OUT_TAIL)])

    return k(feats, src2d, dst2d)


def _tc_body(feats_ref, a0_ref, a1_ref, w_ref, b_ref, out_ref):
    rst = feats_ref[...] + a0_ref[...] + a1_ref[...]
    y = lax.dot_general(rst, w_ref[...], (((1,), (1,)), ((), ())),
                        precision=lax.Precision.HIGHEST,
                        preferred_element_type=jnp.float32)
    y = y + b_ref[...]
    out_ref[...] = jnp.where(y > 0, y, jnp.exp(y) - 1.0)


def _tc_finish(feats, agg, W, b):
    BLK = 2000
    grid = (N // BLK,)
    b2 = b.reshape(1, D)
    return pl.pallas_call(
        _tc_body,
        grid=grid,
        in_specs=[
            pl.BlockSpec((BLK, D), lambda i: (i, 0)),
            pl.BlockSpec((BLK, D), lambda i: (i, 0)),
            pl.BlockSpec((BLK, D), lambda i: (i, 0)),
            pl.BlockSpec((D, D), lambda i: (0, 0)),
            pl.BlockSpec((1, D), lambda i: (0, 0)),
        ],
        out_specs=pl.BlockSpec((BLK, D), lambda i: (i, 0)),
        out_shape=jax.ShapeDtypeStruct((N, D), jnp.float32),
    )(feats, agg[0], agg[1], W, b2)


def kernel(feats, edge_index, W, b):
    ei = edge_index.astype(jnp.int32)
    pad = E_PAD - E
    src = jnp.concatenate([ei[0], jnp.zeros((pad,), jnp.int32)])
    dst = jnp.concatenate([ei[1], jnp.full((pad,), N, jnp.int32)])
    src2d = src.reshape(NW * K, CHUNK)
    dst2d = dst.reshape(NW * K, CHUNK)
    agg = _sc_aggregate(feats, src2d, dst2d)
    return _tc_finish(feats, agg, W, b)


# speedup vs baseline: 1.0327x; 1.0327x over previous
import functools

import jax
import jax.numpy as jnp
from jax import lax
from jax.experimental import pallas as pl
from jax.experimental.pallas import tpu as pltpu
from jax.experimental.pallas import tpu_sc as plsc

N = 10000
D = 128
E = 320000

NUM_CORES = 2
NUM_SUBCORES = 16
NW = NUM_CORES * NUM_SUBCORES

CHUNK = 128
K = 80
KB = 16
NBLK = K // KB
E_PAD = NW * K * CHUNK
NPAD = 10240
ROWS_PER_SUB = NPAD // NUM_SUBCORES
OUT_ROWS_PER_SUB = 624
OUT_TAIL = N - NUM_SUBCORES * OUT_ROWS_PER_SUB


def _sc_aggregate(feats, src2d, dst2d):
    mesh = plsc.VectorSubcoreMesh(core_axis_name="core",
                                  subcore_axis_name="subcore")

    @functools.partial(
        pl.kernel,
        out_type=jax.ShapeDtypeStruct((NUM_CORES, N, D), jnp.float32),
        mesh=mesh,
        scratch_types=[
            pltpu.VMEM((KB, CHUNK), jnp.int32),
            pltpu.VMEM((KB, CHUNK), jnp.int32),
            pltpu.VMEM((CHUNK, D), jnp.float32),
            pltpu.VMEM((CHUNK, D), jnp.float32),
            pltpu.VMEM_SHARED((NPAD, D), jnp.float32),
            pltpu.SemaphoreType.DMA,
            pltpu.SemaphoreType.DMA,
        ],
    )
    def k(feats_hbm, src_hbm, dst_hbm, out_hbm,
          src_v, dst_v, rows0_v, rows1_v, agg_sh, gsem0, gsem1):
        c = lax.axis_index("core")
        s = lax.axis_index("subcore")
        wid = c * NUM_SUBCORES + s

        @pl.loop(0, CHUNK)
        def _(i):
            @pl.loop(0, D, step=16)
            def _(j):
                rows0_v[i, pl.ds(j, 16)] = jnp.zeros((16,), jnp.float32)

        @pl.loop(0, ROWS_PER_SUB // CHUNK)
        def _(t):
            pltpu.sync_copy(rows0_v,
                            agg_sh.at[pl.ds(s * ROWS_PER_SUB + t * CHUNK,
                                            CHUNK)])

        plsc.subcore_barrier()

        @pl.loop(0, NBLK)
        def _(blk):
            base = wid * K + blk * KB
            pltpu.sync_copy(src_hbm.at[pl.ds(base, KB)], src_v)
            pltpu.sync_copy(dst_hbm.at[pl.ds(base, KB)], dst_v)

            pltpu.async_copy(feats_hbm.at[src_v.at[0]], rows0_v, gsem0)
            pltpu.async_copy(feats_hbm.at[src_v.at[1]], rows1_v, gsem1)

            @pl.loop(0, KB, step=2)
            def _(j):
                pltpu.make_async_copy(feats_hbm.at[src_v.at[j]],
                                      rows0_v, gsem0).wait()
                pltpu.sync_copy(rows0_v, agg_sh.at[dst_v.at[j]], add=True)

                @pl.when(j < KB - 2)
                def _():
                    pltpu.async_copy(feats_hbm.at[src_v.at[j + 2]],
                                     rows0_v, gsem0)

                pltpu.make_async_copy(feats_hbm.at[src_v.at[j + 1]],
                                      rows1_v, gsem1).wait()
                pltpu.sync_copy(rows1_v, agg_sh.at[dst_v.at[j + 1]],
                                add=True)

                @pl.when(j < KB - 3)
                def _():
                    pltpu.async_copy(feats_hbm.at[src_v.at[j + 3]],
                                     rows1_v, gsem1)

        plsc.subcore_barrier()

        pltpu.sync_copy(agg_sh.at[pl.ds(s * OUT_ROWS_PER_SUB,
                                        OUT_ROWS_PER_SUB)],
                        out_hbm.at[c, pl.ds(s * OUT_ROWS_PER_SUB,
                                            OUT_ROWS_PER_SUB)])

        @pl.when(s == 0)
        def _():
            pltpu.sync_copy(
                agg_sh.at[pl.ds(NUM_SUBCORES * OUT_ROWS_PER_SUB, OUT_TAIL)],
                out_hbm.at[c, pl.ds(NUM_SUBCORES * OUT_ROWS_PER_SUB,
                                    OUT_TAIL)])

    return k(feats, src2d, dst2d)


def _tc_body(feats_ref, a0_ref, a1_ref, w_ref, b_ref, out_ref):
    rst = feats_ref[...] + a0_ref[...] + a1_ref[...]
    y = lax.dot_general(rst, w_ref[...], (((1,), (1,)), ((), ())),
                        precision=lax.Precision.HIGHEST,
                        preferred_element_type=jnp.float32)
    y = y + b_ref[...]
    out_ref[...] = jnp.where(y > 0, y, jnp.exp(y) - 1.0)


def _tc_finish(feats, agg, W, b):
    BLK = 2000
    grid = (N // BLK,)
    b2 = b.reshape(1, D)
    return pl.pallas_call(
        _tc_body,
        grid=grid,
        in_specs=[
            pl.BlockSpec((BLK, D), lambda i: (i, 0)),
            pl.BlockSpec((BLK, D), lambda i: (i, 0)),
            pl.BlockSpec((BLK, D), lambda i: (i, 0)),
            pl.BlockSpec((D, D), lambda i: (0, 0)),
            pl.BlockSpec((1, D), lambda i: (0, 0)),
        ],
        out_specs=pl.BlockSpec((BLK, D), lambda i: (i, 0)),
        out_shape=jax.ShapeDtypeStruct((N, D), jnp.float32),
    )(feats, agg[0], agg[1], W, b2)


def kernel(feats, edge_index, W, b):
    ei = edge_index.astype(jnp.int32)
    pad = E_PAD - E
    src = jnp.concatenate([ei[0], jnp.zeros((pad,), jnp.int32)])
    dst = jnp.concatenate([ei[1], jnp.full((pad,), N, jnp.int32)])
    src2d = src.reshape(NW * K, CHUNK)
    dst2d = dst.reshape(NW * K, CHUNK)
    agg = _sc_aggregate(feats, src2d, dst2d)
    return _tc_finish(feats, agg, W, b)
